# baseline (speedup 1.0000x reference)
import jax
import jax.numpy as jnp
from jax import lax
from jax.experimental import pallas as pl
from jax.experimental.pallas import tpu as pltpu
from jax.experimental.pallas import tpu_sc as plsc

N, S, D, H, E, K = 1, 2048, 1024, 2048, 8, 2
T = N * S
BLK = 256
NB = (K * T + (BLK - 1) * E) // BLK
GT = NB * BLK
NW = 32
CHG = T // NW
CHC = 16
L = 16


def _route_body(x_ref, wg_ref, bg_ref, pos_ref, wexp_ref, be_ref):
    logits = lax.dot_general(
        wg_ref[...], x_ref[...],
        dimension_numbers=(((1,), (1,)), ((), ())),
        preferred_element_type=jnp.float32,
    ) + bg_ref[...][:, None]
    e_iota = lax.broadcasted_iota(jnp.int32, (E, T), 0)
    m1 = jnp.max(logits, axis=0, keepdims=True)
    am1 = jnp.min(jnp.where(logits == m1, e_iota, E), axis=0, keepdims=True)
    masked = jnp.where(e_iota == am1, -jnp.inf, logits)
    m2 = jnp.max(masked, axis=0, keepdims=True)
    am2 = jnp.min(jnp.where(masked == m2, e_iota, E), axis=0, keepdims=True)
    bexp = jnp.exp(m2 - m1)
    w1 = 1.0 / (1.0 + bexp)
    w2 = bexp / (1.0 + bexp)

    oh1 = (e_iota == am1)
    oh2 = (e_iota == am2)
    bt = (oh1 | oh2).astype(jnp.bfloat16)
    r_iota = lax.broadcasted_iota(jnp.int32, (T, T), 0)
    c_iota = lax.broadcasted_iota(jnp.int32, (T, T), 1)
    tri = (r_iota < c_iota).astype(jnp.bfloat16)
    rank = lax.dot_general(
        bt, tri, dimension_numbers=(((1,), (0,)), ((), ())),
        preferred_element_type=jnp.float32)
    counts = jnp.sum(bt.astype(jnp.float32), axis=1).astype(jnp.int32)
    padded = ((counts + (BLK - 1)) // BLK) * BLK
    ee_r = lax.broadcasted_iota(jnp.int32, (E, E), 0)
    ee_c = lax.broadcasted_iota(jnp.int32, (E, E), 1)
    cum_incl = jnp.sum(jnp.where(ee_c <= ee_r, padded[None, :], 0), axis=1)
    offs = (cum_incl - padded).astype(jnp.float32)

    slot = offs[:, None] + rank
    pos1 = jnp.sum(jnp.where(oh1, slot, 0.0), axis=0, keepdims=True)
    pos2 = jnp.sum(jnp.where(oh2, slot, 0.0), axis=0, keepdims=True)
    pos_ref[...] = jnp.concatenate([pos1, pos2], axis=0).astype(jnp.int32)
    wexp_ref[...] = jnp.broadcast_to(
        jnp.concatenate([w1, w2], axis=0)[:, :, None], (K, T, L))

    bn = lax.broadcasted_iota(jnp.int32, (E, NB + 1), 1) * BLK
    be = jnp.sum((cum_incl[:, None] <= bn).astype(jnp.int32), axis=0)
    nvalid = cum_incl[E - 1] // BLK
    be = jnp.minimum(be, E - 1)
    idx_nb = lax.broadcasted_iota(jnp.int32, (NB + 1,), 0)
    be_ref[...] = jnp.where(idx_nb == NB, nvalid, be)


def _scatter_body(x_hbm, pos_hbm, gx_hbm, rows_a, rows_b,
                  i0a, i0b, i1a, i1b, sem_i, sem_r, sem_s):
    wid = lax.axis_index("s") * 2 + lax.axis_index("c")
    base = wid * CHG
    h = CHG // 2
    ci = [pltpu.async_copy(pos_hbm.at[0, pl.ds(base, h)], i0a, sem_i),
          pltpu.async_copy(pos_hbm.at[0, pl.ds(base + h, h)], i0b, sem_i),
          pltpu.async_copy(pos_hbm.at[1, pl.ds(base, h)], i1a, sem_i),
          pltpu.async_copy(pos_hbm.at[1, pl.ds(base + h, h)], i1b, sem_i)]
    cra = pltpu.async_copy(x_hbm.at[pl.ds(base, h)], rows_a, sem_r)
    crb = pltpu.async_copy(x_hbm.at[pl.ds(base + h, h)], rows_b, sem_r)
    for c in ci:
        c.wait()
    cra.wait()
    sa0 = pltpu.async_copy(rows_a, gx_hbm.at[i0a], sem_s)
    sa1 = pltpu.async_copy(rows_a, gx_hbm.at[i1a], sem_s)
    crb.wait()
    sb0 = pltpu.async_copy(rows_b, gx_hbm.at[i0b], sem_s)
    sb1 = pltpu.async_copy(rows_b, gx_hbm.at[i1b], sem_s)
    sa0.wait()
    sa1.wait()
    sb0.wait()
    sb1.wait()


def _ffn_body(be_ref, gx_ref, w0_ref, b0_ref, w1_ref, b1_ref, w2_ref,
              b2_ref, gy_ref):
    @pl.when(pl.program_id(0) < be_ref[NB])
    def _compute():
        xb = gx_ref[...].astype(jnp.bfloat16)
        h0 = lax.dot_general(
            xb, w0_ref[0].astype(jnp.bfloat16), (((1,), (1,)), ((), ())),
            preferred_element_type=jnp.float32) + b0_ref[0]
        h1 = lax.dot_general(
            xb, w1_ref[0].astype(jnp.bfloat16), (((1,), (1,)), ((), ())),
            preferred_element_type=jnp.float32) + b1_ref[0]
        act = (h0 * (h1 * jax.nn.sigmoid(h1))).astype(jnp.bfloat16)
        gy_ref[...] = lax.dot_general(
            act, w2_ref[0].astype(jnp.bfloat16), (((1,), (1,)), ((), ())),
            preferred_element_type=jnp.float32) + b2_ref[0]


def _combine_body(gy_hbm, pos_hbm, wexp_hbm, out_hbm,
                  r0s, r1s, i0s, i1s, w0s, w1s, sem_g):
    wid = lax.axis_index("s") * 2 + lax.axis_index("c")
    np_ = CHG // CHC

    def issue(p):
        b = p % 2
        base = wid * CHG + p * CHC
        pltpu.sync_copy(pos_hbm.at[0, pl.ds(base, CHC)], i0s[b])
        pltpu.sync_copy(pos_hbm.at[1, pl.ds(base, CHC)], i1s[b])
        pltpu.sync_copy(wexp_hbm.at[0, pl.ds(base, CHC)], w0s[b])
        pltpu.sync_copy(wexp_hbm.at[1, pl.ds(base, CHC)], w1s[b])
        g0 = pltpu.async_copy(gy_hbm.at[i0s[b]], r0s[b], sem_g)
        g1 = pltpu.async_copy(gy_hbm.at[i1s[b]], r1s[b], sem_g)
        return g0, g1

    pending = issue(0)
    for p in range(np_):
        b = p % 2
        base = wid * CHG + p * CHC
        nxt = issue(p + 1) if p + 1 < np_ else None
        pending[0].wait()
        pending[1].wait()
        r0_v, r1_v, w0_v, w1_v = r0s[b], r1s[b], w0s[b], w1s[b]

        def body(i, _):
            wa = w0_v[i, :]
            wb = w1_v[i, :]
            for j in range(D // L):
                sl = pl.ds(j * L, L)
                r0_v[i, sl] = r0_v[i, sl] * wa + r1_v[i, sl] * wb
            return 0

        lax.fori_loop(0, CHC, body, 0)
        pltpu.sync_copy(r0_v, out_hbm.at[pl.ds(base, CHC)])
        pending = nxt


def kernel(x, Wg, bg, W0, b0, W1, b1, W2, b2):
    x2 = x.reshape(T, D)

    pos, wexp, be = pl.pallas_call(
        _route_body,
        out_shape=[
            jax.ShapeDtypeStruct((K, T), jnp.int32),
            jax.ShapeDtypeStruct((K, T, L), jnp.float32),
            jax.ShapeDtypeStruct((NB + 1,), jnp.int32),
        ],
        in_specs=[
            pl.BlockSpec((T, D), lambda: (0, 0)),
            pl.BlockSpec((E, D), lambda: (0, 0)),
            pl.BlockSpec((E,), lambda: (0,)),
        ],
        out_specs=[
            pl.BlockSpec((K, T), lambda: (0, 0)),
            pl.BlockSpec((K, T, L), lambda: (0, 0, 0)),
            pl.BlockSpec((NB + 1,), lambda: (0,)),
        ],
    )(x2, Wg, bg)

    gx = pl.kernel(
        _scatter_body,
        out_type=jax.ShapeDtypeStruct((GT, D), jnp.float32),
        mesh=plsc.VectorSubcoreMesh(core_axis_name="c",
                                    subcore_axis_name="s"),
        scratch_types=[
            pltpu.VMEM((CHG // 2, D), jnp.float32),
            pltpu.VMEM((CHG // 2, D), jnp.float32),
            pltpu.VMEM((CHG // 2,), jnp.int32),
            pltpu.VMEM((CHG // 2,), jnp.int32),
            pltpu.VMEM((CHG // 2,), jnp.int32),
            pltpu.VMEM((CHG // 2,), jnp.int32),
            pltpu.SemaphoreType.DMA,
            pltpu.SemaphoreType.DMA,
            pltpu.SemaphoreType.DMA,
        ],
    )(x2, pos)

    gy = pl.pallas_call(
        _ffn_body,
        grid_spec=pltpu.PrefetchScalarGridSpec(
            num_scalar_prefetch=1,
            grid=(NB,),
            in_specs=[
                pl.BlockSpec((BLK, D),
                             lambda b, be: (jnp.where(b < be[NB], b, 0), 0)),
                pl.BlockSpec((1, H, D), lambda b, be: (be[b], 0, 0)),
                pl.BlockSpec((1, 1, H), lambda b, be: (be[b], 0, 0)),
                pl.BlockSpec((1, H, D), lambda b, be: (be[b], 0, 0)),
                pl.BlockSpec((1, 1, H), lambda b, be: (be[b], 0, 0)),
                pl.BlockSpec((1, D, H), lambda b, be: (be[b], 0, 0)),
                pl.BlockSpec((1, 1, D), lambda b, be: (be[b], 0, 0)),
            ],
            out_specs=pl.BlockSpec((BLK, D), lambda b, be: (b, 0)),
        ),
        out_shape=jax.ShapeDtypeStruct((GT, D), jnp.float32),
        compiler_params=pltpu.CompilerParams(
            dimension_semantics=("arbitrary",),
        ),
    )(be, gx, W0, b0.reshape(E, 1, H),
      W1, b1.reshape(E, 1, H),
      W2, b2.reshape(E, 1, D))

    out = pl.kernel(
        _combine_body,
        out_type=jax.ShapeDtypeStruct((T, D), jnp.float32),
        mesh=plsc.VectorSubcoreMesh(core_axis_name="c",
                                    subcore_axis_name="s"),
        scratch_types=[
            [pltpu.VMEM((CHC, D), jnp.float32) for _ in range(2)],
            [pltpu.VMEM((CHC, D), jnp.float32) for _ in range(2)],
            [pltpu.VMEM((CHC,), jnp.int32) for _ in range(2)],
            [pltpu.VMEM((CHC,), jnp.int32) for _ in range(2)],
            [pltpu.VMEM((CHC, L), jnp.float32) for _ in range(2)],
            [pltpu.VMEM((CHC, L), jnp.float32) for _ in range(2)],
            pltpu.SemaphoreType.DMA,
        ],
    )(gy, pos, wexp)

    return out.reshape(N, S, D)


# speedup vs baseline: 1.0295x; 1.0295x over previous
import jax
import jax.numpy as jnp
from jax import lax
from jax.experimental import pallas as pl
from jax.experimental.pallas import tpu as pltpu
from jax.experimental.pallas import tpu_sc as plsc

N, S, D, H, E, K = 1, 2048, 1024, 2048, 8, 2
T = N * S
BLK = 256
NB = (K * T + (BLK - 1) * E) // BLK
GT = NB * BLK
NW = 32
CHG = T // NW
CHC = 16
L = 16


def _route_body(x_ref, wg_ref, bg_ref, pos_ref, wexp_ref, be_ref):
    logits = lax.dot_general(
        wg_ref[...], x_ref[...],
        dimension_numbers=(((1,), (1,)), ((), ())),
        preferred_element_type=jnp.float32,
    ) + bg_ref[...][:, None]
    e_iota = lax.broadcasted_iota(jnp.int32, (E, T), 0)
    m1 = jnp.max(logits, axis=0, keepdims=True)
    am1 = jnp.min(jnp.where(logits == m1, e_iota, E), axis=0, keepdims=True)
    masked = jnp.where(e_iota == am1, -jnp.inf, logits)
    m2 = jnp.max(masked, axis=0, keepdims=True)
    am2 = jnp.min(jnp.where(masked == m2, e_iota, E), axis=0, keepdims=True)
    bexp = jnp.exp(m2 - m1)
    w1 = 1.0 / (1.0 + bexp)
    w2 = bexp / (1.0 + bexp)

    oh1 = (e_iota == am1)
    oh2 = (e_iota == am2)
    bt = (oh1 | oh2).astype(jnp.bfloat16)
    r_iota = lax.broadcasted_iota(jnp.int32, (T, T), 0)
    c_iota = lax.broadcasted_iota(jnp.int32, (T, T), 1)
    tri = (r_iota < c_iota).astype(jnp.bfloat16)
    rank = lax.dot_general(
        bt, tri, dimension_numbers=(((1,), (0,)), ((), ())),
        preferred_element_type=jnp.float32)
    counts = jnp.sum(bt.astype(jnp.float32), axis=1).astype(jnp.int32)
    padded = ((counts + (BLK - 1)) // BLK) * BLK
    ee_r = lax.broadcasted_iota(jnp.int32, (E, E), 0)
    ee_c = lax.broadcasted_iota(jnp.int32, (E, E), 1)
    cum_incl = jnp.sum(jnp.where(ee_c <= ee_r, padded[None, :], 0), axis=1)
    offs = (cum_incl - padded).astype(jnp.float32)

    slot = offs[:, None] + rank
    pos1 = jnp.sum(jnp.where(oh1, slot, 0.0), axis=0, keepdims=True)
    pos2 = jnp.sum(jnp.where(oh2, slot, 0.0), axis=0, keepdims=True)
    pos_ref[...] = jnp.concatenate([pos1, pos2], axis=0).astype(jnp.int32)
    wexp_ref[...] = jnp.broadcast_to(
        jnp.concatenate([w1, w2], axis=0)[:, :, None], (K, T, L))

    bn = lax.broadcasted_iota(jnp.int32, (E, NB + 1), 1) * BLK
    be = jnp.sum((cum_incl[:, None] <= bn).astype(jnp.int32), axis=0)
    nvalid = cum_incl[E - 1] // BLK
    be = jnp.minimum(be, E - 1)
    idx_nb = lax.broadcasted_iota(jnp.int32, (NB + 1,), 0)
    be_ref[...] = jnp.where(idx_nb == NB, nvalid, be)


def _scatter_body(x_hbm, pos_hbm, gx_hbm, rows_a, rows_b,
                  i0a, i0b, i1a, i1b, sem_i, sem_r, sem_s):
    wid = lax.axis_index("s") * 2 + lax.axis_index("c")
    base = wid * CHG
    h = CHG // 2
    ci = [pltpu.async_copy(pos_hbm.at[0, pl.ds(base, h)], i0a, sem_i),
          pltpu.async_copy(pos_hbm.at[0, pl.ds(base + h, h)], i0b, sem_i),
          pltpu.async_copy(pos_hbm.at[1, pl.ds(base, h)], i1a, sem_i),
          pltpu.async_copy(pos_hbm.at[1, pl.ds(base + h, h)], i1b, sem_i)]
    cra = pltpu.async_copy(x_hbm.at[pl.ds(base, h)], rows_a, sem_r)
    crb = pltpu.async_copy(x_hbm.at[pl.ds(base + h, h)], rows_b, sem_r)
    for c in ci:
        c.wait()
    cra.wait()
    sa0 = pltpu.async_copy(rows_a, gx_hbm.at[i0a], sem_s)
    sa1 = pltpu.async_copy(rows_a, gx_hbm.at[i1a], sem_s)
    crb.wait()
    sb0 = pltpu.async_copy(rows_b, gx_hbm.at[i0b], sem_s)
    sb1 = pltpu.async_copy(rows_b, gx_hbm.at[i1b], sem_s)
    sa0.wait()
    sa1.wait()
    sb0.wait()
    sb1.wait()


def _ffn_body(be_ref, gx_ref, w0_ref, b0_ref, w1_ref, b1_ref, w2_ref,
              b2_ref, gy_ref):
    @pl.when(pl.program_id(0) < be_ref[NB])
    def _compute():
        xb = gx_ref[...].astype(jnp.bfloat16)
        h0 = lax.dot_general(
            xb, w0_ref[0].astype(jnp.bfloat16), (((1,), (1,)), ((), ())),
            preferred_element_type=jnp.float32) + b0_ref[0]
        h1 = lax.dot_general(
            xb, w1_ref[0].astype(jnp.bfloat16), (((1,), (1,)), ((), ())),
            preferred_element_type=jnp.float32) + b1_ref[0]
        act = (h0 * (h1 * jax.nn.sigmoid(h1))).astype(jnp.bfloat16)
        gy_ref[...] = lax.dot_general(
            act, w2_ref[0].astype(jnp.bfloat16), (((1,), (1,)), ((), ())),
            preferred_element_type=jnp.float32) + b2_ref[0]


def _combine_body(gy_hbm, pos_hbm, wexp_hbm, out_hbm,
                  r0s, r1s, i0_v, i1_v, w0_v, w1_v, sem_i, sem_g):
    wid = lax.axis_index("s") * 2 + lax.axis_index("c")
    base = wid * CHG
    np_ = CHG // CHC
    ci = [pltpu.async_copy(pos_hbm.at[0, pl.ds(base, CHG)], i0_v, sem_i),
          pltpu.async_copy(pos_hbm.at[1, pl.ds(base, CHG)], i1_v, sem_i),
          pltpu.async_copy(wexp_hbm.at[0, pl.ds(base, CHG)], w0_v, sem_i),
          pltpu.async_copy(wexp_hbm.at[1, pl.ds(base, CHG)], w1_v, sem_i)]
    for c in ci:
        c.wait()

    def issue(p):
        b = p % 2
        sl = pl.ds(p * CHC, CHC)
        g0 = pltpu.async_copy(gy_hbm.at[i0_v.at[sl]], r0s[b], sem_g)
        g1 = pltpu.async_copy(gy_hbm.at[i1_v.at[sl]], r1s[b], sem_g)
        return g0, g1

    pending = issue(0)
    for p in range(np_):
        b = p % 2
        nxt = issue(p + 1) if p + 1 < np_ else None
        pending[0].wait()
        pending[1].wait()
        r0_v, r1_v = r0s[b], r1s[b]

        def body(i, _):
            wa = w0_v[p * CHC + i, :]
            wb = w1_v[p * CHC + i, :]
            for j in range(D // L):
                sl = pl.ds(j * L, L)
                r0_v[i, sl] = r0_v[i, sl] * wa + r1_v[i, sl] * wb
            return 0

        lax.fori_loop(0, CHC, body, 0)
        pltpu.sync_copy(r0_v, out_hbm.at[pl.ds(base + p * CHC, CHC)])
        pending = nxt


def kernel(x, Wg, bg, W0, b0, W1, b1, W2, b2):
    x2 = x.reshape(T, D)

    pos, wexp, be = pl.pallas_call(
        _route_body,
        out_shape=[
            jax.ShapeDtypeStruct((K, T), jnp.int32),
            jax.ShapeDtypeStruct((K, T, L), jnp.float32),
            jax.ShapeDtypeStruct((NB + 1,), jnp.int32),
        ],
        in_specs=[
            pl.BlockSpec((T, D), lambda: (0, 0)),
            pl.BlockSpec((E, D), lambda: (0, 0)),
            pl.BlockSpec((E,), lambda: (0,)),
        ],
        out_specs=[
            pl.BlockSpec((K, T), lambda: (0, 0)),
            pl.BlockSpec((K, T, L), lambda: (0, 0, 0)),
            pl.BlockSpec((NB + 1,), lambda: (0,)),
        ],
    )(x2, Wg, bg)

    gx = pl.kernel(
        _scatter_body,
        out_type=jax.ShapeDtypeStruct((GT, D), jnp.float32),
        mesh=plsc.VectorSubcoreMesh(core_axis_name="c",
                                    subcore_axis_name="s"),
        scratch_types=[
            pltpu.VMEM((CHG // 2, D), jnp.float32),
            pltpu.VMEM((CHG // 2, D), jnp.float32),
            pltpu.VMEM((CHG // 2,), jnp.int32),
            pltpu.VMEM((CHG // 2,), jnp.int32),
            pltpu.VMEM((CHG // 2,), jnp.int32),
            pltpu.VMEM((CHG // 2,), jnp.int32),
            pltpu.SemaphoreType.DMA,
            pltpu.SemaphoreType.DMA,
            pltpu.SemaphoreType.DMA,
        ],
    )(x2, pos)

    gy = pl.pallas_call(
        _ffn_body,
        grid_spec=pltpu.PrefetchScalarGridSpec(
            num_scalar_prefetch=1,
            grid=(NB,),
            in_specs=[
                pl.BlockSpec((BLK, D),
                             lambda b, be: (jnp.where(b < be[NB], b, 0), 0)),
                pl.BlockSpec((1, H, D), lambda b, be: (be[b], 0, 0)),
                pl.BlockSpec((1, 1, H), lambda b, be: (be[b], 0, 0)),
                pl.BlockSpec((1, H, D), lambda b, be: (be[b], 0, 0)),
                pl.BlockSpec((1, 1, H), lambda b, be: (be[b], 0, 0)),
                pl.BlockSpec((1, D, H), lambda b, be: (be[b], 0, 0)),
                pl.BlockSpec((1, 1, D), lambda b, be: (be[b], 0, 0)),
            ],
            out_specs=pl.BlockSpec((BLK, D), lambda b, be: (b, 0)),
        ),
        out_shape=jax.ShapeDtypeStruct((GT, D), jnp.float32),
        compiler_params=pltpu.CompilerParams(
            dimension_semantics=("arbitrary",),
        ),
    )(be, gx, W0, b0.reshape(E, 1, H),
      W1, b1.reshape(E, 1, H),
      W2, b2.reshape(E, 1, D))

    out = pl.kernel(
        _combine_body,
        out_type=jax.ShapeDtypeStruct((T, D), jnp.float32),
        mesh=plsc.VectorSubcoreMesh(core_axis_name="c",
                                    subcore_axis_name="s"),
        scratch_types=[
            [pltpu.VMEM((CHC, D), jnp.float32) for _ in range(2)],
            [pltpu.VMEM((CHC, D), jnp.float32) for _ in range(2)],
            pltpu.VMEM((CHG,), jnp.int32),
            pltpu.VMEM((CHG,), jnp.int32),
            pltpu.VMEM((CHG, L), jnp.float32),
            pltpu.VMEM((CHG, L), jnp.float32),
            pltpu.SemaphoreType.DMA,
            pltpu.SemaphoreType.DMA,
        ],
    )(gy, pos, wexp)

    return out.reshape(N, S, D)
